# baseline (speedup 1.0000x reference)
import functools
import math

import numpy as np
import jax
import jax.numpy as jnp
from jax.experimental import pallas as pl

_B = 32
_H = 16
_L = 256
_DM = 4096
_K = int(2 * math.log(_L))
_NF = 136

_f = np.arange(_NF)
_t = np.arange(_L)
_ang = 2.0 * np.pi * np.outer(_f, _t) / _L
_mask = (_f <= _L // 2).astype(np.float64)[:, None]
_FWD = np.concatenate(
    [np.cos(_ang) * _mask, np.sin(_ang) * _mask], axis=0
).astype(np.float32)
_w = np.where((_f == 0) | (_f == _L // 2), 1.0, 2.0) * _mask[:, 0]
_angi = 2.0 * np.pi * np.outer(_t, _f) / _L
_INV = np.concatenate(
    [np.cos(_angi) * _w / _L, -np.sin(_angi) * _w / _L], axis=1
).astype(np.float32)


def _corr_topk_kernel(fwd_ref, inv_ref, q_ref, k_ref, v_ref, o_ref):
    fwd = fwd_ref[...]
    inv = inv_ref[...]
    q = q_ref[0]
    k = k_ref[0]
    v = v_ref[0]

    dot = functools.partial(jnp.dot, preferred_element_type=jnp.float32,
                            precision=jax.lax.Precision.HIGHEST)
    qk = jnp.concatenate([q, k], axis=1)
    tf = dot(fwd, qk)
    qr = tf[:_NF, :_L]
    qs = tf[_NF:, :_L]
    kr = tf[:_NF, _L:]
    ks = tf[_NF:, _L:]
    pr = qr * kr + qs * ks
    pi = qr * ks - qs * kr
    p = jnp.concatenate([pr, pi], axis=0)
    corr = dot(inv, p)

    riota = jax.lax.broadcasted_iota(jnp.int32, (_L, _L), 0)
    diota = jax.lax.broadcasted_iota(jnp.int32, (1, _L), 1)

    c = corr
    m0 = None
    num = jnp.zeros((1, _L), dtype=jnp.float32)
    den = jnp.zeros((1, _L), dtype=jnp.float32)
    neg_inf = jnp.float32(-jnp.inf)
    for i in range(_K):
        m = jnp.max(c, axis=0, keepdims=True)
        is_m = c == m
        pos = jnp.min(jnp.where(is_m, riota, _L), axis=0, keepdims=True)
        sel = riota == pos
        c = jnp.where(sel, neg_inf, c)
        if i == 0:
            m0 = m
            e = jnp.ones((1, _L), dtype=jnp.float32)
        else:
            e = jnp.exp(m - m0)
        tgt = jnp.minimum(pos + diota, _L - 1)
        onehot = riota == tgt
        g = jnp.sum(jnp.where(onehot, v, 0.0), axis=0, keepdims=True)
        num = num + e * g
        den = den + e

    out_row = (2.0 * _L) * num / den
    o_ref[0] = jnp.broadcast_to(out_row, (_L, _L))


def kernel(Q, K, V):
    grid = (_B, _H)
    bh_spec = pl.BlockSpec((1, _L, _L), lambda b, h: (b, 0, h))
    out = pl.pallas_call(
        _corr_topk_kernel,
        grid=grid,
        in_specs=[
            pl.BlockSpec((2 * _NF, _L), lambda b, h: (0, 0)),
            pl.BlockSpec((_L, 2 * _NF), lambda b, h: (0, 0)),
            bh_spec, bh_spec, bh_spec,
        ],
        out_specs=bh_spec,
        out_shape=jax.ShapeDtypeStruct((_B, _L, _DM), jnp.float32),
    )(jnp.asarray(_FWD), jnp.asarray(_INV), Q, K, V)
    return out


# speedup vs baseline: 1.4172x; 1.4172x over previous
import functools
import math

import numpy as np
import jax
import jax.numpy as jnp
from jax.experimental import pallas as pl

_B = 32
_H = 16
_L = 256
_DM = 4096
_K = int(2 * math.log(_L))
_NF = 136

_f = np.arange(_NF)
_t = np.arange(_L)
_ang = 2.0 * np.pi * np.outer(_f, _t) / _L
_mask = (_f <= _L // 2).astype(np.float64)[:, None]
_FWD = np.concatenate(
    [np.cos(_ang) * _mask, np.sin(_ang) * _mask], axis=0
).astype(np.float32)
_w = np.where((_f == 0) | (_f == _L // 2), 1.0, 2.0) * _mask[:, 0]
_angi = 2.0 * np.pi * np.outer(_t, _f) / _L
_INVC = (np.cos(_angi) * _w / _L).astype(np.float32)
_INVS = (-np.sin(_angi) * _w / _L).astype(np.float32)


def _corr_topk_kernel(fwd_ref, invc_ref, invs_ref, q_ref, k_ref, v_ref, o_ref):
    fwd = fwd_ref[...]
    invc = invc_ref[...]
    invs = invs_ref[...]
    q = q_ref[0]
    k = k_ref[0]
    v = v_ref[0]

    dot = functools.partial(jnp.dot, preferred_element_type=jnp.float32)

    def split(x):
        hi = x.astype(jnp.bfloat16)
        lo = (x - hi.astype(jnp.float32)).astype(jnp.bfloat16)
        return hi, lo

    def dot3(a, x):
        ah, al = split(a)
        xh, xl = split(x)
        return dot(ah, xh) + (dot(ah, xl) + dot(al, xh))

    qf = dot3(fwd, q)
    kf = dot3(fwd, k)
    qr = qf[:_NF]
    qs = qf[_NF:]
    kr = kf[:_NF]
    ks = kf[_NF:]
    pr = qr * kr + qs * ks
    pi = qr * ks - qs * kr
    corr = dot3(invc, pr) + dot3(invs, pi)

    riota = jax.lax.broadcasted_iota(jnp.int32, (_L, _L), 0)
    diota = jax.lax.broadcasted_iota(jnp.int32, (1, _L), 1)

    c = corr
    m0 = None
    num = jnp.zeros((1, _L), dtype=jnp.float32)
    den = jnp.zeros((1, _L), dtype=jnp.float32)
    neg_inf = jnp.float32(-jnp.inf)
    for i in range(_K):
        m = jnp.max(c, axis=0, keepdims=True)
        is_m = c == m
        pos = jnp.min(jnp.where(is_m, riota, _L), axis=0, keepdims=True)
        sel = riota == pos
        c = jnp.where(sel, neg_inf, c)
        if i == 0:
            m0 = m
            e = jnp.ones((1, _L), dtype=jnp.float32)
        else:
            e = jnp.exp(m - m0)
        tgt = jnp.minimum(pos + diota, _L - 1)
        onehot = riota == tgt
        g = jnp.sum(jnp.where(onehot, v, 0.0), axis=0, keepdims=True)
        num = num + e * g
        den = den + e

    out_row = (2.0 * _L) * num / den
    o_ref[0] = jnp.broadcast_to(out_row, (_L, _L))


def kernel(Q, K, V):
    grid = (_B, _H)
    bh_spec = pl.BlockSpec((1, _L, _L), lambda b, h: (b, 0, h))
    out = pl.pallas_call(
        _corr_topk_kernel,
        grid=grid,
        in_specs=[
            pl.BlockSpec((2 * _NF, _L), lambda b, h: (0, 0)),
            pl.BlockSpec((_L, _NF), lambda b, h: (0, 0)),
            pl.BlockSpec((_L, _NF), lambda b, h: (0, 0)),
            bh_spec, bh_spec, bh_spec,
        ],
        out_specs=bh_spec,
        out_shape=jax.ShapeDtypeStruct((_B, _L, _DM), jnp.float32),
    )(jnp.asarray(_FWD), jnp.asarray(_INVC), jnp.asarray(_INVS), Q, K, V)
    return out


# speedup vs baseline: 1.7392x; 1.2272x over previous
import functools
import math

import numpy as np
import jax
import jax.numpy as jnp
from jax.experimental import pallas as pl

_B = 32
_H = 16
_L = 256
_DM = 4096
_K = int(2 * math.log(_L))
_NF = 136

_f = np.arange(_NF)
_t = np.arange(_L)
_ang = 2.0 * np.pi * np.outer(_f, _t) / _L
_mask = (_f <= _L // 2).astype(np.float64)[:, None]
_FWD = np.concatenate(
    [np.cos(_ang) * _mask, np.sin(_ang) * _mask], axis=0
).astype(np.float32)
_w = np.where((_f == 0) | (_f == _L // 2), 1.0, 2.0) * _mask[:, 0]
_angi = 2.0 * np.pi * np.outer(_t, _f) / _L
_INVC = (np.cos(_angi) * _w / _L).astype(np.float32)
_INVS = (-np.sin(_angi) * _w / _L).astype(np.float32)
_angp = 2.0 * np.pi * np.outer(_f, _t) / _L
_PHC = (np.cos(_angp) * _mask).astype(np.float32)
_PHS = (np.sin(_angp) * _mask).astype(np.float32)


def _corr_topk_kernel(fwd_ref, invc_ref, invs_ref, phc_ref, phs_ref,
                      q_ref, k_ref, v_ref, o_ref):
    fwd = fwd_ref[...]
    invc = invc_ref[...]
    invs = invs_ref[...]
    phc = phc_ref[...]
    phs = phs_ref[...]
    q = q_ref[0]
    k = k_ref[0]
    v = v_ref[0]

    dot = functools.partial(jnp.dot, preferred_element_type=jnp.float32)

    def split(x):
        hi = x.astype(jnp.bfloat16)
        lo = (x - hi.astype(jnp.float32)).astype(jnp.bfloat16)
        return hi, lo

    def dot3(a, x):
        ah, al = split(a)
        xh, xl = split(x)
        return dot(ah, xh) + (dot(ah, xl) + dot(al, xh))

    qf = dot3(fwd, q)
    kf = dot3(fwd, k)
    qr = qf[:_NF]
    qs = qf[_NF:]
    kr = kf[:_NF]
    ks = kf[_NF:]
    pr = qr * kr + qs * ks
    pi = qr * ks - qs * kr
    prs = pr * phc + pi * phs
    pis = pi * phc - pr * phs
    c = dot3(invc, prs) + dot3(invs, pis)

    riota = jax.lax.broadcasted_iota(jnp.int32, (_L, _L), 0)
    diota = jax.lax.broadcasted_iota(jnp.int32, (_L, _L), 1)
    vmod = jnp.where(riota >= diota, v, jnp.broadcast_to(v[_L - 1:_L], (_L, _L)))

    m0 = None
    num = jnp.zeros((1, _L), dtype=jnp.float32)
    den = jnp.zeros((1, _L), dtype=jnp.float32)
    neg_inf = jnp.float32(-jnp.inf)
    for i in range(_K):
        m = jnp.max(c, axis=0, keepdims=True)
        is_m = c == m
        g = jnp.sum(jnp.where(is_m, vmod, 0.0), axis=0, keepdims=True)
        c = jnp.where(is_m, neg_inf, c)
        if i == 0:
            m0 = m
            e = jnp.ones((1, _L), dtype=jnp.float32)
        else:
            e = jnp.exp(m - m0)
        num = num + e * g
        den = den + e

    out_row = (2.0 * _L) * num / den
    o_ref[0] = jnp.broadcast_to(out_row, (_L, _L))


def kernel(Q, K, V):
    grid = (_B, _H)
    bh_spec = pl.BlockSpec((1, _L, _L), lambda b, h: (b, 0, h))
    out = pl.pallas_call(
        _corr_topk_kernel,
        grid=grid,
        in_specs=[
            pl.BlockSpec((2 * _NF, _L), lambda b, h: (0, 0)),
            pl.BlockSpec((_L, _NF), lambda b, h: (0, 0)),
            pl.BlockSpec((_L, _NF), lambda b, h: (0, 0)),
            pl.BlockSpec((_NF, _L), lambda b, h: (0, 0)),
            pl.BlockSpec((_NF, _L), lambda b, h: (0, 0)),
            bh_spec, bh_spec, bh_spec,
        ],
        out_specs=bh_spec,
        out_shape=jax.ShapeDtypeStruct((_B, _L, _DM), jnp.float32),
    )(jnp.asarray(_FWD), jnp.asarray(_INVC), jnp.asarray(_INVS),
      jnp.asarray(_PHC), jnp.asarray(_PHS), Q, K, V)
    return out


# speedup vs baseline: 1.9595x; 1.1267x over previous
import functools
import math

import numpy as np
import jax
import jax.numpy as jnp
from jax.experimental import pallas as pl
from jax.experimental.pallas import tpu as pltpu

_B = 32
_H = 16
_L = 256
_DM = 4096
_K = int(2 * math.log(_L))
_NF = 136

_f = np.arange(_NF)
_t = np.arange(_L)
_ang = 2.0 * np.pi * np.outer(_f, _t) / _L
_mask = (_f <= _L // 2).astype(np.float64)[:, None]
_FWD = np.concatenate(
    [np.cos(_ang) * _mask, np.sin(_ang) * _mask], axis=0
).astype(np.float32)
_w = np.where((_f == 0) | (_f == _L // 2), 1.0, 2.0) * _mask[:, 0]
_angi = 2.0 * np.pi * np.outer(_t, _f) / _L
_INVC = (np.cos(_angi) * _w / _L).astype(np.float32)
_INVS = (-np.sin(_angi) * _w / _L).astype(np.float32)
_angp = 2.0 * np.pi * np.outer(_f, _t) / _L
_PHC = (np.cos(_angp) * _mask).astype(np.float32)
_PHS = (np.sin(_angp) * _mask).astype(np.float32)


def _corr_topk_kernel(fwd_ref, invc_ref, invs_ref, phc_ref, phs_ref,
                      q_ref, k_ref, v_ref, o_ref, corr_ref):
    ph = jax.lax.rem(pl.program_id(0), 2)
    fwd = fwd_ref[...]
    invc = invc_ref[...]
    invs = invs_ref[...]
    phc = phc_ref[...]
    phs = phs_ref[...]
    q = q_ref[0]
    k = k_ref[0]
    v = v_ref[0]
    c = corr_ref[1 - ph]

    dot = functools.partial(jnp.dot, preferred_element_type=jnp.float32)

    def split(x):
        hi = x.astype(jnp.bfloat16)
        lo = (x - hi.astype(jnp.float32)).astype(jnp.bfloat16)
        return hi, lo

    def dot3(a, x):
        ah, al = split(a)
        xh, xl = split(x)
        return dot(ah, xh) + (dot(ah, xl) + dot(al, xh))

    qf = dot3(fwd, q)
    kf = dot3(fwd, k)
    qr = qf[:_NF]
    qs = qf[_NF:]
    kr = kf[:_NF]
    ks = kf[_NF:]
    pr = qr * kr + qs * ks
    pi = qr * ks - qs * kr
    prs = pr * phc + pi * phs
    pis = pi * phc - pr * phs
    corr_ref[ph] = dot3(invc, prs) + dot3(invs, pis)

    riota = jax.lax.broadcasted_iota(jnp.int32, (_L, _L), 0)
    diota = jax.lax.broadcasted_iota(jnp.int32, (_L, _L), 1)
    vmod = jnp.where(riota >= diota, v, jnp.broadcast_to(v[_L - 1:_L], (_L, _L)))

    m0 = None
    num = jnp.zeros((1, _L), dtype=jnp.float32)
    den = jnp.zeros((1, _L), dtype=jnp.float32)
    neg_inf = jnp.float32(-jnp.inf)
    for i in range(_K):
        m = jnp.max(c, axis=0, keepdims=True)
        is_m = c == m
        g = jnp.sum(jnp.where(is_m, vmod, 0.0), axis=0, keepdims=True)
        c = jnp.where(is_m, neg_inf, c)
        if i == 0:
            m0 = m
            e = jnp.ones((1, _L), dtype=jnp.float32)
        else:
            e = jnp.exp(m - m0)
        num = num + e * g
        den = den + e

    out_row = (2.0 * _L) * num / den
    o_ref[0] = jnp.broadcast_to(out_row, (_L, _L))


def kernel(Q, K, V):
    nb = _B * _H

    def cur_spec(s):
        sc = jnp.minimum(s, nb - 1)
        return (sc // _H, 0, jax.lax.rem(sc, _H))

    def prev_spec(s):
        sp = jnp.maximum(s - 1, 0)
        return (sp // _H, 0, jax.lax.rem(sp, _H))

    qk_spec = pl.BlockSpec((1, _L, _L), cur_spec)
    vo_spec = pl.BlockSpec((1, _L, _L), prev_spec)
    out = pl.pallas_call(
        _corr_topk_kernel,
        grid=(nb + 1,),
        in_specs=[
            pl.BlockSpec((2 * _NF, _L), lambda s: (0, 0)),
            pl.BlockSpec((_L, _NF), lambda s: (0, 0)),
            pl.BlockSpec((_L, _NF), lambda s: (0, 0)),
            pl.BlockSpec((_NF, _L), lambda s: (0, 0)),
            pl.BlockSpec((_NF, _L), lambda s: (0, 0)),
            qk_spec, qk_spec, vo_spec,
        ],
        out_specs=vo_spec,
        out_shape=jax.ShapeDtypeStruct((_B, _L, _DM), jnp.float32),
        scratch_shapes=[pltpu.VMEM((2, _L, _L), jnp.float32)],
    )(jnp.asarray(_FWD), jnp.asarray(_INVC), jnp.asarray(_INVS),
      jnp.asarray(_PHC), jnp.asarray(_PHS), Q, K, V)
    return out


# speedup vs baseline: 2.2792x; 1.1631x over previous
import functools
import math

import numpy as np
import jax
import jax.numpy as jnp
from jax.experimental import pallas as pl
from jax.experimental.pallas import tpu as pltpu

_B = 32
_H = 16
_L = 256
_DM = 4096
_K = int(2 * math.log(_L))
_NF = 136

_f = np.arange(_NF)
_t = np.arange(_L)
_ang = 2.0 * np.pi * np.outer(_f, _t) / _L
_mask = (_f <= _L // 2).astype(np.float64)[:, None]
_FWD = np.concatenate(
    [np.cos(_ang) * _mask, np.sin(_ang) * _mask], axis=0
).astype(np.float32)
_w = np.where((_f == 0) | (_f == _L // 2), 1.0, 2.0) * _mask[:, 0]
_angi = 2.0 * np.pi * np.outer(_t, _f) / _L
_INVC = (np.cos(_angi) * _w / _L).astype(np.float32)
_INVS = (-np.sin(_angi) * _w / _L).astype(np.float32)
_angp = 2.0 * np.pi * np.outer(_f, _t) / _L
_PHC = (np.cos(_angp) * _mask).astype(np.float32)
_PHS = (np.sin(_angp) * _mask).astype(np.float32)


def _const_split(x):
    hi = jnp.asarray(x).astype(jnp.bfloat16)
    lo = (jnp.asarray(x) - hi.astype(jnp.float32)).astype(jnp.bfloat16)
    return hi, lo


def _corr_topk_kernel(fwdh_ref, fwdl_ref, invch_ref, invcl_ref,
                      invsh_ref, invsl_ref, phc_ref, phs_ref,
                      q_ref, k_ref, v_ref, o_ref, corr_ref):
    ph = jax.lax.rem(pl.program_id(0), 2)
    q = q_ref[0]
    k = k_ref[0]
    v = v_ref[0]
    c0 = corr_ref[1 - ph]

    dot = functools.partial(jnp.dot, preferred_element_type=jnp.float32)

    def split(x):
        hi = x.astype(jnp.bfloat16)
        lo = (x - hi.astype(jnp.float32)).astype(jnp.bfloat16)
        return hi, lo

    def dot3(ah, al, x):
        xh, xl = split(x)
        return dot(ah, xh) + (dot(ah, xl) + dot(al, xh))

    qf = dot3(fwdh_ref[...], fwdl_ref[...], q)
    kf = dot3(fwdh_ref[...], fwdl_ref[...], k)
    qr = qf[:_NF]
    qs = qf[_NF:]
    kr = kf[:_NF]
    ks = kf[_NF:]
    pr = qr * kr + qs * ks
    pi = qr * ks - qs * kr
    prs = pr * phc_ref[...] + pi * phs_ref[...]
    pis = pi * phc_ref[...] - pr * phs_ref[...]
    corr_ref[ph] = (dot3(invch_ref[...], invcl_ref[...], prs)
                    + dot3(invsh_ref[...], invsl_ref[...], pis))

    riota = jax.lax.broadcasted_iota(jnp.int32, (_L, _L), 0)
    diota = jax.lax.broadcasted_iota(jnp.int32, (_L, _L), 1)
    vmod = jnp.where(riota >= diota, v, jnp.broadcast_to(v[_L - 1:_L], (_L, _L)))

    neg_inf = jnp.float32(-jnp.inf)
    c = c0
    m0 = None
    for i in range(_K):
        m = jnp.max(c, axis=0, keepdims=True)
        if i == 0:
            m0 = m
        c = jnp.where(c == m, neg_inf, c)

    wexp = jnp.exp(c0 - m0)
    w = jnp.where(c == neg_inf, wexp, 0.0)
    den = jnp.sum(w, axis=0, keepdims=True)
    num = jnp.sum(w * vmod, axis=0, keepdims=True)
    out_row = (2.0 * _L) * num / den
    o_ref[0] = jnp.broadcast_to(out_row, (_L, _L))


def kernel(Q, K, V):
    nb = _B * _H

    def cur_spec(s):
        sc = jnp.minimum(s, nb - 1)
        return (sc // _H, 0, jax.lax.rem(sc, _H))

    def prev_spec(s):
        sp = jnp.maximum(s - 1, 0)
        return (sp // _H, 0, jax.lax.rem(sp, _H))

    const2 = lambda s: (0, 0)
    qk_spec = pl.BlockSpec((1, _L, _L), cur_spec)
    vo_spec = pl.BlockSpec((1, _L, _L), prev_spec)
    fwdh, fwdl = _const_split(_FWD)
    invch, invcl = _const_split(_INVC)
    invsh, invsl = _const_split(_INVS)
    out = pl.pallas_call(
        _corr_topk_kernel,
        grid=(nb + 1,),
        in_specs=[
            pl.BlockSpec((2 * _NF, _L), const2),
            pl.BlockSpec((2 * _NF, _L), const2),
            pl.BlockSpec((_L, _NF), const2),
            pl.BlockSpec((_L, _NF), const2),
            pl.BlockSpec((_L, _NF), const2),
            pl.BlockSpec((_L, _NF), const2),
            pl.BlockSpec((_NF, _L), const2),
            pl.BlockSpec((_NF, _L), const2),
            qk_spec, qk_spec, vo_spec,
        ],
        out_specs=vo_spec,
        out_shape=jax.ShapeDtypeStruct((_B, _L, _DM), jnp.float32),
        scratch_shapes=[pltpu.VMEM((2, _L, _L), jnp.float32)],
    )(fwdh, fwdl, invch, invcl, invsh, invsl,
      jnp.asarray(_PHC), jnp.asarray(_PHS), Q, K, V)
    return out


# speedup vs baseline: 3.0224x; 1.3261x over previous
import functools
import math

import numpy as np
import jax
import jax.numpy as jnp
from jax.experimental import pallas as pl
from jax.experimental.pallas import tpu as pltpu

_B = 32
_H = 16
_L = 256
_DM = 4096
_K = int(2 * math.log(_L))
_NF = 136

_f = np.arange(_NF)
_t = np.arange(_L)
_ang = 2.0 * np.pi * np.outer(_f, _t) / _L
_mask = (_f <= _L // 2).astype(np.float64)[:, None]
_FWD = np.concatenate(
    [np.cos(_ang) * _mask, np.sin(_ang) * _mask], axis=0
).astype(np.float32)
_w = np.where((_f == 0) | (_f == _L // 2), 1.0, 2.0) * _mask[:, 0]
_angi = 2.0 * np.pi * np.outer(_t, _f) / _L
_INVC = (np.cos(_angi) * _w / _L).astype(np.float32)
_INVS = (-np.sin(_angi) * _w / _L).astype(np.float32)
_angp = 2.0 * np.pi * np.outer(_f, _t) / _L
_PHC = np.tile((np.cos(_angp) * _mask).astype(np.float32), (1, 2))
_PHS = np.tile((np.sin(_angp) * _mask).astype(np.float32), (1, 2))
_W = 2 * _L


def _const_split(x):
    hi = jnp.asarray(x).astype(jnp.bfloat16)
    lo = (jnp.asarray(x) - hi.astype(jnp.float32)).astype(jnp.bfloat16)
    return hi, lo


def _corr_topk_kernel(fwdh_ref, fwdl_ref, invch_ref, invcl_ref,
                      invsh_ref, invsl_ref, phc_ref, phs_ref,
                      q_ref, k_ref, v_ref, o_ref, corr_ref):
    ph = jax.lax.rem(pl.program_id(0), 2)
    q = q_ref[0]
    k = k_ref[0]
    v = v_ref[0]
    c0 = corr_ref[1 - ph]

    dot = functools.partial(jnp.dot, preferred_element_type=jnp.float32)

    def split(x):
        hi = x.astype(jnp.bfloat16)
        lo = (x - hi.astype(jnp.float32)).astype(jnp.bfloat16)
        return hi, lo

    def dot3(ah, al, x):
        xh, xl = split(x)
        return dot(ah, xh) + (dot(ah, xl) + dot(al, xh))

    qf = dot3(fwdh_ref[...], fwdl_ref[...], q)
    kf = dot3(fwdh_ref[...], fwdl_ref[...], k)
    qr = qf[:_NF]
    qs = qf[_NF:]
    kr = kf[:_NF]
    ks = kf[_NF:]
    pr = qr * kr + qs * ks
    pi = qr * ks - qs * kr
    prs = pr * phc_ref[...] + pi * phs_ref[...]
    pis = pi * phc_ref[...] - pr * phs_ref[...]
    corr_ref[ph] = (dot3(invch_ref[...], invcl_ref[...], prs)
                    + dot3(invsh_ref[...], invsl_ref[...], pis))

    riota = jax.lax.broadcasted_iota(jnp.int32, (_L, _W), 0)
    diota = jax.lax.broadcasted_iota(jnp.int32, (_L, _W), 1) & (_L - 1)
    vmod = jnp.where(riota >= diota, v, jnp.broadcast_to(v[_L - 1:_L], (_L, _W)))

    neg_inf = jnp.float32(-jnp.inf)
    c = c0
    m0 = None
    for i in range(_K):
        m = jnp.max(c, axis=0, keepdims=True)
        if i == 0:
            m0 = m
        c = jnp.where(c == m, neg_inf, c)

    wexp = jnp.exp(c0 - m0)
    w = jnp.where(c == neg_inf, wexp, 0.0)
    den = jnp.sum(w, axis=0, keepdims=True)
    num = jnp.sum(w * vmod, axis=0, keepdims=True)
    out_row = (2.0 * _L) * num / den
    o_ref[0] = jnp.broadcast_to(out_row, (_L, _W))


def kernel(Q, K, V):
    nh = _H // 2
    nb = _B * nh

    def cur_spec(s):
        sc = jnp.minimum(s, nb - 1)
        return (sc // nh, 0, jax.lax.rem(sc, nh))

    def prev_spec(s):
        sp = jnp.maximum(s - 1, 0)
        return (sp // nh, 0, jax.lax.rem(sp, nh))

    const2 = lambda s: (0, 0)
    qk_spec = pl.BlockSpec((1, _L, _W), cur_spec)
    vo_spec = pl.BlockSpec((1, _L, _W), prev_spec)
    fwdh, fwdl = _const_split(_FWD)
    invch, invcl = _const_split(_INVC)
    invsh, invsl = _const_split(_INVS)
    out = pl.pallas_call(
        _corr_topk_kernel,
        grid=(nb + 1,),
        in_specs=[
            pl.BlockSpec((2 * _NF, _L), const2),
            pl.BlockSpec((2 * _NF, _L), const2),
            pl.BlockSpec((_L, _NF), const2),
            pl.BlockSpec((_L, _NF), const2),
            pl.BlockSpec((_L, _NF), const2),
            pl.BlockSpec((_L, _NF), const2),
            pl.BlockSpec((_NF, _W), const2),
            pl.BlockSpec((_NF, _W), const2),
            qk_spec, qk_spec, vo_spec,
        ],
        out_specs=vo_spec,
        out_shape=jax.ShapeDtypeStruct((_B, _L, _DM), jnp.float32),
        scratch_shapes=[pltpu.VMEM((2, _L, _W), jnp.float32)],
    )(fwdh, fwdl, invch, invcl, invsh, invsl,
      jnp.asarray(_PHC), jnp.asarray(_PHS), Q, K, V)
    return out


# speedup vs baseline: 3.1747x; 1.0504x over previous
import functools
import math

import numpy as np
import jax
import jax.numpy as jnp
from jax.experimental import pallas as pl
from jax.experimental.pallas import tpu as pltpu

_B = 32
_H = 16
_L = 256
_DM = 4096
_K = int(2 * math.log(_L))
_NF = 136

_f = np.arange(_NF)
_t = np.arange(_L)
_ang = 2.0 * np.pi * np.outer(_f, _t) / _L
_mask = (_f <= _L // 2).astype(np.float64)[:, None]
_FWD = np.concatenate(
    [np.cos(_ang) * _mask, np.sin(_ang) * _mask], axis=0
).astype(np.float32)
_w = np.where((_f == 0) | (_f == _L // 2), 1.0, 2.0) * _mask[:, 0]
_angi = 2.0 * np.pi * np.outer(_t, _f) / _L
_INVC = (np.cos(_angi) * _w / _L).astype(np.float32)
_INVS = (-np.sin(_angi) * _w / _L).astype(np.float32)
_angp = 2.0 * np.pi * np.outer(_f, _t) / _L
_PHC = np.tile((np.cos(_angp) * _mask).astype(np.float32), (1, 4))
_PHS = np.tile((np.sin(_angp) * _mask).astype(np.float32), (1, 4))
_W = 4 * _L


def _const_split(x):
    hi = jnp.asarray(x).astype(jnp.bfloat16)
    lo = (jnp.asarray(x) - hi.astype(jnp.float32)).astype(jnp.bfloat16)
    return hi, lo


def _corr_topk_kernel(fwdh_ref, fwdl_ref, invch_ref, invcl_ref,
                      invsh_ref, invsl_ref, phc_ref, phs_ref,
                      q_ref, k_ref, v_ref, o_ref, corr_ref):
    ph = jax.lax.rem(pl.program_id(0), 2)
    q = q_ref[0]
    k = k_ref[0]
    v = v_ref[0]
    c0 = corr_ref[1 - ph]

    dot = functools.partial(jnp.dot, preferred_element_type=jnp.float32)

    def split(x):
        hi = x.astype(jnp.bfloat16)
        lo = (x - hi.astype(jnp.float32)).astype(jnp.bfloat16)
        return hi, lo

    def dot3(ah, al, x):
        xh, xl = split(x)
        return dot(ah, xh) + (dot(ah, xl) + dot(al, xh))

    qf = dot3(fwdh_ref[...], fwdl_ref[...], q)
    kf = dot3(fwdh_ref[...], fwdl_ref[...], k)
    qr = qf[:_NF]
    qs = qf[_NF:]
    kr = kf[:_NF]
    ks = kf[_NF:]
    pr = qr * kr + qs * ks
    pi = qr * ks - qs * kr
    prs = pr * phc_ref[...] + pi * phs_ref[...]
    pis = pi * phc_ref[...] - pr * phs_ref[...]
    corr_ref[ph] = (dot3(invch_ref[...], invcl_ref[...], prs)
                    + dot3(invsh_ref[...], invsl_ref[...], pis))

    riota = jax.lax.broadcasted_iota(jnp.int32, (_L, _W), 0)
    diota = jax.lax.broadcasted_iota(jnp.int32, (_L, _W), 1) & (_L - 1)
    vmod = jnp.where(riota >= diota, v, jnp.broadcast_to(v[_L - 1:_L], (_L, _W)))

    neg_inf = jnp.float32(-jnp.inf)
    c = c0
    m0 = None
    for i in range(_K):
        m = jnp.max(c, axis=0, keepdims=True)
        if i == 0:
            m0 = m
        c = jnp.where(c == m, neg_inf, c)

    wexp = jnp.exp(c0 - m0)
    w = jnp.where(c == neg_inf, wexp, 0.0)
    den = jnp.sum(w, axis=0, keepdims=True)
    num = jnp.sum(w * vmod, axis=0, keepdims=True)
    out_row = (2.0 * _L) * num / den
    o_ref[0] = jnp.broadcast_to(out_row, (_L, _W))


def kernel(Q, K, V):
    nh = _H // 4
    nb = _B * nh

    def cur_spec(s):
        sc = jnp.minimum(s, nb - 1)
        return (sc // nh, 0, jax.lax.rem(sc, nh))

    def prev_spec(s):
        sp = jnp.maximum(s - 1, 0)
        return (sp // nh, 0, jax.lax.rem(sp, nh))

    const2 = lambda s: (0, 0)
    qk_spec = pl.BlockSpec((1, _L, _W), cur_spec)
    vo_spec = pl.BlockSpec((1, _L, _W), prev_spec)
    fwdh, fwdl = _const_split(_FWD)
    invch, invcl = _const_split(_INVC)
    invsh, invsl = _const_split(_INVS)
    out = pl.pallas_call(
        _corr_topk_kernel,
        grid=(nb + 1,),
        in_specs=[
            pl.BlockSpec((2 * _NF, _L), const2),
            pl.BlockSpec((2 * _NF, _L), const2),
            pl.BlockSpec((_L, _NF), const2),
            pl.BlockSpec((_L, _NF), const2),
            pl.BlockSpec((_L, _NF), const2),
            pl.BlockSpec((_L, _NF), const2),
            pl.BlockSpec((_NF, _W), const2),
            pl.BlockSpec((_NF, _W), const2),
            qk_spec, qk_spec, vo_spec,
        ],
        out_specs=vo_spec,
        out_shape=jax.ShapeDtypeStruct((_B, _L, _DM), jnp.float32),
        scratch_shapes=[pltpu.VMEM((2, _L, _W), jnp.float32)],
    )(fwdh, fwdl, invch, invcl, invsh, invsl,
      jnp.asarray(_PHC), jnp.asarray(_PHS), Q, K, V)
    return out


# speedup vs baseline: 3.2444x; 1.0220x over previous
import functools
import math

import numpy as np
import jax
import jax.numpy as jnp
from jax.experimental import pallas as pl
from jax.experimental.pallas import tpu as pltpu

_B = 32
_H = 16
_L = 256
_DM = 4096
_K = int(2 * math.log(_L))
_NF = 136

_f = np.arange(_NF)
_t = np.arange(_L)
_ang = 2.0 * np.pi * np.outer(_f, _t) / _L
_mask = (_f <= _L // 2).astype(np.float64)[:, None]
_FWD = np.concatenate(
    [np.cos(_ang) * _mask, np.sin(_ang) * _mask], axis=0
).astype(np.float32)
_w = np.where((_f == 0) | (_f == _L // 2), 1.0, 2.0) * _mask[:, 0]
_angi = 2.0 * np.pi * np.outer(_t, _f) / _L
_INVC = (np.cos(_angi) * _w / _L).astype(np.float32)
_INVS = (-np.sin(_angi) * _w / _L).astype(np.float32)
_angp = 2.0 * np.pi * np.outer(_f, _t) / _L
_PHC = np.tile((np.cos(_angp) * _mask).astype(np.float32), (1, 4))
_PHS = np.tile((np.sin(_angp) * _mask).astype(np.float32), (1, 4))
_W = 4 * _L


def _const_split(x):
    hi = jnp.asarray(x).astype(jnp.bfloat16)
    lo = (jnp.asarray(x) - hi.astype(jnp.float32)).astype(jnp.bfloat16)
    return hi, lo


def _corr_topk_kernel(fwdh_ref, fwdl_ref, invch_ref, invcl_ref,
                      invsh_ref, invsl_ref, phc_ref, phs_ref,
                      q_ref, k_ref, v_ref, o_ref, corr_ref):
    ph = jax.lax.rem(pl.program_id(0), 2)
    q = q_ref[0]
    k = k_ref[0]
    v = v_ref[0]
    c0 = corr_ref[1 - ph]

    dot = functools.partial(jnp.dot, preferred_element_type=jnp.float32)

    def split(x):
        hi = x.astype(jnp.bfloat16)
        lo = (x - hi.astype(jnp.float32)).astype(jnp.bfloat16)
        return hi, lo

    def dot3(ah, al, x):
        xh, xl = split(x)
        return dot(ah, xh) + (dot(ah, xl) + dot(al, xh))

    qf = dot3(fwdh_ref[...], fwdl_ref[...], q)
    kf = dot3(fwdh_ref[...], fwdl_ref[...], k)
    qr = qf[:_NF]
    qs = qf[_NF:]
    kr = kf[:_NF]
    ks = kf[_NF:]
    pr = qr * kr + qs * ks
    pi = qr * ks - qs * kr
    prs = pr * phc_ref[...] + pi * phs_ref[...]
    pis = pi * phc_ref[...] - pr * phs_ref[...]
    corr_ref[ph] = (dot3(invch_ref[...], invcl_ref[...], prs)
                    + dot3(invsh_ref[...], invsl_ref[...], pis))

    riota = jax.lax.broadcasted_iota(jnp.int32, (_L, _W), 0)
    diota = jax.lax.broadcasted_iota(jnp.int32, (_L, _W), 1) & (_L - 1)
    vmod = jnp.where(riota >= diota, v, jnp.broadcast_to(v[_L - 1:_L], (_L, _W)))

    neg_inf = jnp.float32(-jnp.inf)
    m0 = jnp.max(c0, axis=0, keepdims=True)
    m = m0
    for _ in range(_K - 1):
        m = jnp.max(jnp.where(c0 < m, c0, neg_inf), axis=0, keepdims=True)

    wexp = jnp.exp(c0 - m0)
    w = jnp.where(c0 >= m, wexp, 0.0)
    den = jnp.sum(w, axis=0, keepdims=True)
    num = jnp.sum(w * vmod, axis=0, keepdims=True)
    out_row = (2.0 * _L) * num / den
    o_ref[0] = jnp.broadcast_to(out_row, (_L, _W))


def kernel(Q, K, V):
    nh = _H // 4
    nb = _B * nh

    def cur_spec(s):
        sc = jnp.minimum(s, nb - 1)
        return (sc // nh, 0, jax.lax.rem(sc, nh))

    def prev_spec(s):
        sp = jnp.maximum(s - 1, 0)
        return (sp // nh, 0, jax.lax.rem(sp, nh))

    const2 = lambda s: (0, 0)
    qk_spec = pl.BlockSpec((1, _L, _W), cur_spec)
    vo_spec = pl.BlockSpec((1, _L, _W), prev_spec)
    fwdh, fwdl = _const_split(_FWD)
    invch, invcl = _const_split(_INVC)
    invsh, invsl = _const_split(_INVS)
    out = pl.pallas_call(
        _corr_topk_kernel,
        grid=(nb + 1,),
        in_specs=[
            pl.BlockSpec((2 * _NF, _L), const2),
            pl.BlockSpec((2 * _NF, _L), const2),
            pl.BlockSpec((_L, _NF), const2),
            pl.BlockSpec((_L, _NF), const2),
            pl.BlockSpec((_L, _NF), const2),
            pl.BlockSpec((_L, _NF), const2),
            pl.BlockSpec((_NF, _W), const2),
            pl.BlockSpec((_NF, _W), const2),
            qk_spec, qk_spec, vo_spec,
        ],
        out_specs=vo_spec,
        out_shape=jax.ShapeDtypeStruct((_B, _L, _DM), jnp.float32),
        scratch_shapes=[pltpu.VMEM((2, _L, _W), jnp.float32)],
    )(fwdh, fwdl, invch, invcl, invsh, invsl,
      jnp.asarray(_PHC), jnp.asarray(_PHS), Q, K, V)
    return out
